# Initial kernel scaffold; baseline (speedup 1.0000x reference)
import jax
import jax.numpy as jnp
from jax.experimental import pallas as pl


def kernel(hidden, beliefs, active_mask, W_obs, w1, b1, w2, b2, w3, b3, w4, b4):
    raise NotImplementedError("write your pallas kernel here")


# speedup vs baseline: 1.1689x; 1.1689x over previous
import functools

import jax
import jax.numpy as jnp
from jax import lax
from jax.experimental import pallas as pl
from jax.experimental.pallas import tpu as pltpu

EPSILON = 1e-6
MATCH_THRESHOLD = 0.5
RADIUS_THRESHOLD = 0.05

NB = 512


def _fused_kernel(hid_ref, wall_ref, wbd_ref, b13_ref, b24_ref, belT_ref,
                  maskbias_ref, obsb_ref, slots_ref, simsout_ref,
                  angsT_ref):
    i = pl.program_id(0)
    S = belT_ref.shape[1]

    @pl.when(i == 0)
    def _init():
        belT = belT_ref[...]
        norm = jnp.sqrt(jnp.sum(belT * belT, axis=0, keepdims=True))
        angsT_ref[...] = (belT / jnp.maximum(norm, EPSILON)).astype(jnp.bfloat16)

    hb = hid_ref[...].astype(jnp.bfloat16)
    acc = jnp.dot(hb, wall_ref[...], preferred_element_type=jnp.float32)
    obs = acc[:, :256]
    h13 = jnp.maximum(acc[:, 256:] + b13_ref[...], 0.0)
    gl = jnp.dot(h13.astype(jnp.bfloat16), wbd_ref[...],
                 preferred_element_type=jnp.float32) + b24_ref[...]
    gate = jax.nn.sigmoid(gl[:, 0:1])
    prec = jax.nn.softplus(gl[:, 1:2])
    gp = gate * prec

    onorm = jnp.sqrt(jnp.sum(obs * obs, axis=1, keepdims=True))
    ang = obs / jnp.maximum(onorm, EPSILON)
    obsb_ref[...] = ang * gp

    sims = jnp.dot(ang.astype(jnp.bfloat16), angsT_ref[...],
                   preferred_element_type=jnp.float32)
    sims = sims + maskbias_ref[...]
    best = jnp.max(sims, axis=1, keepdims=True)
    iota = lax.broadcasted_iota(jnp.int32, sims.shape, 1)
    cand = jnp.where(sims == best, iota, S)
    bidx = jnp.min(cand, axis=1)
    bestv = best[:, 0]
    matched = (gp[:, 0] > RADIUS_THRESHOLD) & (bestv > MATCH_THRESHOLD)
    slots_ref[0, 0, :] = jnp.where(matched, bidx, -1).astype(jnp.int32)
    simsout_ref[0, 0, :] = jnp.where(matched, bestv, 0.0)


@functools.partial(jax.jit, static_argnames=())
def kernel(hidden, beliefs, active_mask, W_obs, w1, b1, w2, b2, w3, b3, w4, b4):
    B, T, H = hidden.shape
    D = W_obs.shape[0]
    Hq = w1.shape[0]
    S = beliefs.shape[0]
    N = B * T
    nblk = N // NB

    hid2d = hidden.reshape(N, H)
    wall = jnp.concatenate([W_obs.T, w1.T, w3.T], axis=1).astype(jnp.bfloat16)
    wbd = jnp.zeros((2 * Hq, 2), jnp.float32)
    wbd = wbd.at[:Hq, 0].set(w2[0]).at[Hq:, 1].set(w4[0]).astype(jnp.bfloat16)
    b13 = jnp.concatenate([b1, b3]).reshape(1, 2 * Hq).astype(jnp.float32)
    b24 = jnp.concatenate([b2, b4]).reshape(1, 2).astype(jnp.float32)
    belT = beliefs.T
    maskbias = jnp.where(active_mask, 0.0, -jnp.inf).reshape(1, S).astype(jnp.float32)

    grid = (nblk,)
    obsb, slots3, sims3 = pl.pallas_call(
        _fused_kernel,
        grid=grid,
        in_specs=[
            pl.BlockSpec((NB, H), lambda i: (i, 0)),
            pl.BlockSpec((H, D + 2 * Hq), lambda i: (0, 0)),
            pl.BlockSpec((2 * Hq, 2), lambda i: (0, 0)),
            pl.BlockSpec((1, 2 * Hq), lambda i: (0, 0)),
            pl.BlockSpec((1, 2), lambda i: (0, 0)),
            pl.BlockSpec((D, S), lambda i: (0, 0)),
            pl.BlockSpec((1, S), lambda i: (0, 0)),
        ],
        out_specs=[
            pl.BlockSpec((NB, D), lambda i: (i, 0)),
            pl.BlockSpec((1, 1, NB), lambda i: (i, 0, 0)),
            pl.BlockSpec((1, 1, NB), lambda i: (i, 0, 0)),
        ],
        out_shape=[
            jax.ShapeDtypeStruct((N, D), jnp.float32),
            jax.ShapeDtypeStruct((nblk, 1, NB), jnp.int32),
            jax.ShapeDtypeStruct((nblk, 1, NB), jnp.float32),
        ],
        scratch_shapes=[pltpu.VMEM((D, S), jnp.bfloat16)],
    )(hid2d, wall, wbd, b13, b24, belT, maskbias)

    return (obsb.reshape(B, T, D), slots3.reshape(N), sims3.reshape(N))


# speedup vs baseline: 1.3906x; 1.1896x over previous
import functools

import jax
import jax.numpy as jnp
from jax import lax
from jax.experimental import pallas as pl
from jax.experimental.pallas import tpu as pltpu

EPSILON = 1e-6
MATCH_THRESHOLD = 0.5
RADIUS_THRESHOLD = 0.05

NB = 512


def _fused_kernel(hid_ref, wall_ref, wbd_ref, b13_ref, b24_ref, belT_ref,
                  maskbias_ref, revcol_ref, obsb_ref, slots_ref, simsout_ref,
                  angsT_ref):
    i = pl.program_id(0)
    S = belT_ref.shape[1]

    @pl.when(i == 0)
    def _init():
        belT = belT_ref[...]
        norm = jnp.sqrt(jnp.sum(belT * belT, axis=0, keepdims=True))
        angsT_ref[...] = (belT / jnp.maximum(norm, EPSILON)).astype(jnp.bfloat16)

    hb = hid_ref[...].astype(jnp.bfloat16)
    acc = jnp.dot(hb, wall_ref[...], preferred_element_type=jnp.float32)
    obs = acc[:, :256]
    h13 = jnp.maximum(acc[:, 256:] + b13_ref[...], 0.0)
    gl = jnp.dot(h13.astype(jnp.bfloat16), wbd_ref[...],
                 preferred_element_type=jnp.float32) + b24_ref[...]
    gate = jax.nn.sigmoid(gl[:, 0:1])
    prec = jax.nn.softplus(gl[:, 1:2])
    gp = gate * prec

    onorm = jnp.sqrt(jnp.sum(obs * obs, axis=1, keepdims=True))
    ang = obs / jnp.maximum(onorm, EPSILON)
    obsb_ref[...] = ang * gp

    sims = jnp.dot(ang.astype(jnp.bfloat16), angsT_ref[...],
                   preferred_element_type=jnp.float32)
    b = lax.bitcast_convert_type(sims + maskbias_ref[...], jnp.int32)
    packed = (b & jnp.int32(-8192)) | revcol_ref[...]
    pbest = jnp.max(packed, axis=1)
    bidx = (S - 1) - (pbest & jnp.int32(8191))
    bestv = lax.bitcast_convert_type(pbest & jnp.int32(-8192),
                                     jnp.float32) - 3.0
    matched = (gp[:, 0] > RADIUS_THRESHOLD) & (bestv > MATCH_THRESHOLD)
    slots_ref[0, 0, :] = jnp.where(matched, bidx, -1).astype(jnp.int32)
    simsout_ref[0, 0, :] = jnp.where(matched, bestv, 0.0)


@functools.partial(jax.jit, static_argnames=())
def kernel(hidden, beliefs, active_mask, W_obs, w1, b1, w2, b2, w3, b3, w4, b4):
    B, T, H = hidden.shape
    D = W_obs.shape[0]
    Hq = w1.shape[0]
    S = beliefs.shape[0]
    N = B * T
    nblk = N // NB

    hid2d = hidden.reshape(N, H)
    wall = jnp.concatenate([W_obs.T, w1.T, w3.T], axis=1).astype(jnp.bfloat16)
    wbd = jnp.zeros((2 * Hq, 2), jnp.float32)
    wbd = wbd.at[:Hq, 0].set(w2[0]).at[Hq:, 1].set(w4[0]).astype(jnp.bfloat16)
    b13 = jnp.concatenate([b1, b3]).reshape(1, 2 * Hq).astype(jnp.float32)
    b24 = jnp.concatenate([b2, b4]).reshape(1, 2).astype(jnp.float32)
    belT = beliefs.T
    maskbias = jnp.where(active_mask, 3.0, -jnp.inf).reshape(1, S).astype(jnp.float32)
    revcol = (S - 1 - jnp.arange(S, dtype=jnp.int32)).reshape(1, S)

    grid = (nblk,)
    obsb, slots3, sims3 = pl.pallas_call(
        _fused_kernel,
        grid=grid,
        in_specs=[
            pl.BlockSpec((NB, H), lambda i: (i, 0)),
            pl.BlockSpec((H, D + 2 * Hq), lambda i: (0, 0)),
            pl.BlockSpec((2 * Hq, 2), lambda i: (0, 0)),
            pl.BlockSpec((1, 2 * Hq), lambda i: (0, 0)),
            pl.BlockSpec((1, 2), lambda i: (0, 0)),
            pl.BlockSpec((D, S), lambda i: (0, 0)),
            pl.BlockSpec((1, S), lambda i: (0, 0)),
            pl.BlockSpec((1, S), lambda i: (0, 0)),
        ],
        out_specs=[
            pl.BlockSpec((NB, D), lambda i: (i, 0)),
            pl.BlockSpec((1, 1, NB), lambda i: (i, 0, 0)),
            pl.BlockSpec((1, 1, NB), lambda i: (i, 0, 0)),
        ],
        out_shape=[
            jax.ShapeDtypeStruct((N, D), jnp.float32),
            jax.ShapeDtypeStruct((nblk, 1, NB), jnp.int32),
            jax.ShapeDtypeStruct((nblk, 1, NB), jnp.float32),
        ],
        scratch_shapes=[pltpu.VMEM((D, S), jnp.bfloat16)],
    )(hid2d, wall, wbd, b13, b24, belT, maskbias, revcol)

    return (obsb.reshape(B, T, D), slots3.reshape(N), sims3.reshape(N))


# speedup vs baseline: 1.5130x; 1.0880x over previous
import functools

import jax
import jax.numpy as jnp
from jax import lax
from jax.experimental import pallas as pl
from jax.experimental.pallas import tpu as pltpu

EPSILON = 1e-6
MATCH_THRESHOLD = 0.5
RADIUS_THRESHOLD = 0.05

NB = 512


def _fused_kernel(hid_ref, wall_ref, wbd_ref, b13_ref, b24_ref, belT_ref,
                  andm_ref, orm_ref, obsb_ref, slots_ref, simsout_ref,
                  angsT_ref):
    i = pl.program_id(0)
    S = belT_ref.shape[1]

    @pl.when(i == 0)
    def _init():
        belT = belT_ref[...]
        norm = jnp.sqrt(jnp.sum(belT * belT, axis=0, keepdims=True))
        angsT_ref[...] = (belT / jnp.maximum(norm, EPSILON)).astype(jnp.bfloat16)

    hb = hid_ref[...].astype(jnp.bfloat16)
    acc = jnp.dot(hb, wall_ref[...], preferred_element_type=jnp.float32)
    obs = acc[:, :256]
    h13 = jnp.maximum(acc[:, 256:] + b13_ref[...], 0.0)
    gl = jnp.dot(h13.astype(jnp.bfloat16), wbd_ref[...],
                 preferred_element_type=jnp.float32) + b24_ref[...]
    gate = jax.nn.sigmoid(gl[:, 0:1])
    prec = jax.nn.softplus(gl[:, 1:2])
    gp = gate * prec

    onorm = jnp.sqrt(jnp.sum(obs * obs, axis=1, keepdims=True))
    rinv = 1.0 / jnp.maximum(onorm, EPSILON)
    obsb_ref[...] = obs * (rinv * gp)

    raw = jnp.dot(obs.astype(jnp.bfloat16), angsT_ref[...],
                  preferred_element_type=jnp.float32)
    b = lax.bitcast_convert_type(raw, jnp.int32)
    packed = (b & andm_ref[...]) | orm_ref[...]
    pmax = jnp.max(lax.bitcast_convert_type(packed, jnp.float32), axis=1)
    pbest = lax.bitcast_convert_type(pmax, jnp.int32)
    bidx = (S - 1) - (pbest & jnp.int32(8191))
    bestv = lax.bitcast_convert_type(pbest & jnp.int32(-8192),
                                     jnp.float32) * rinv[:, 0]
    matched = (gp[:, 0] > RADIUS_THRESHOLD) & (bestv > MATCH_THRESHOLD)
    slots_ref[0, 0, :] = jnp.where(matched, bidx, -1).astype(jnp.int32)
    simsout_ref[0, 0, :] = jnp.where(matched, bestv, 0.0)


@functools.partial(jax.jit, static_argnames=())
def kernel(hidden, beliefs, active_mask, W_obs, w1, b1, w2, b2, w3, b3, w4, b4):
    B, T, H = hidden.shape
    D = W_obs.shape[0]
    Hq = w1.shape[0]
    S = beliefs.shape[0]
    N = B * T
    nblk = N // NB

    hid2d = hidden.reshape(N, H)
    wall = jnp.concatenate([W_obs.T, w1.T, w3.T], axis=1).astype(jnp.bfloat16)
    wbd = jnp.zeros((2 * Hq, 2), jnp.float32)
    wbd = wbd.at[:Hq, 0].set(w2[0]).at[Hq:, 1].set(w4[0]).astype(jnp.bfloat16)
    b13 = jnp.concatenate([b1, b3]).reshape(1, 2 * Hq).astype(jnp.float32)
    b24 = jnp.concatenate([b2, b4]).reshape(1, 2).astype(jnp.float32)
    belT = beliefs.T
    revcol = (S - 1 - jnp.arange(S, dtype=jnp.int32)).reshape(1, S)
    andm = jnp.where(active_mask, jnp.int32(-8192), jnp.int32(0)).reshape(1, S)
    orm = revcol | jnp.where(active_mask, jnp.int32(0),
                             jnp.int32(-2147483648)).reshape(1, S)

    grid = (nblk,)
    obsb, slots3, sims3 = pl.pallas_call(
        _fused_kernel,
        grid=grid,
        in_specs=[
            pl.BlockSpec((NB, H), lambda i: (i, 0)),
            pl.BlockSpec((H, D + 2 * Hq), lambda i: (0, 0)),
            pl.BlockSpec((2 * Hq, 2), lambda i: (0, 0)),
            pl.BlockSpec((1, 2 * Hq), lambda i: (0, 0)),
            pl.BlockSpec((1, 2), lambda i: (0, 0)),
            pl.BlockSpec((D, S), lambda i: (0, 0)),
            pl.BlockSpec((1, S), lambda i: (0, 0)),
            pl.BlockSpec((1, S), lambda i: (0, 0)),
        ],
        out_specs=[
            pl.BlockSpec((NB, D), lambda i: (i, 0)),
            pl.BlockSpec((1, 1, NB), lambda i: (i, 0, 0)),
            pl.BlockSpec((1, 1, NB), lambda i: (i, 0, 0)),
        ],
        out_shape=[
            jax.ShapeDtypeStruct((N, D), jnp.float32),
            jax.ShapeDtypeStruct((nblk, 1, NB), jnp.int32),
            jax.ShapeDtypeStruct((nblk, 1, NB), jnp.float32),
        ],
        scratch_shapes=[pltpu.VMEM((D, S), jnp.bfloat16)],
    )(hid2d, wall, wbd, b13, b24, belT, andm, orm)

    return (obsb.reshape(B, T, D), slots3.reshape(N), sims3.reshape(N))


# speedup vs baseline: 1.7239x; 1.1394x over previous
import functools

import jax
import jax.numpy as jnp
from jax import lax
from jax.experimental import pallas as pl
from jax.experimental.pallas import tpu as pltpu

EPSILON = 1e-6
MATCH_THRESHOLD = 0.5
RADIUS_THRESHOLD = 0.05

NB = 512


def _dot_t(a, b):
    return lax.dot_general(a, b, (((1,), (1,)), ((), ())),
                           preferred_element_type=jnp.float32)


def _fused_kernel(hid_ref, wcat_ref, wbd_ref, b13_ref, b24_ref, bel_ref,
                  andm_ref, orm_ref, obsb_ref, slots_ref, simsout_ref,
                  angs_ref):
    i = pl.program_id(0)
    S = bel_ref.shape[0]

    @pl.when(i == 0)
    def _init():
        bel = bel_ref[...]
        norm = jnp.sqrt(jnp.sum(bel * bel, axis=1, keepdims=True))
        angs_ref[...] = (bel / jnp.maximum(norm, EPSILON)).astype(jnp.bfloat16)

    hb = hid_ref[...].astype(jnp.bfloat16)
    acc = _dot_t(hb, wcat_ref[...])
    obs = acc[:, :256]
    h13 = jnp.maximum(acc[:, 256:] + b13_ref[...], 0.0)
    gl = _dot_t(h13.astype(jnp.bfloat16), wbd_ref[...]) + b24_ref[...]
    gate = jax.nn.sigmoid(gl[:, 0:1])
    prec = jax.nn.softplus(gl[:, 1:2])
    gp = gate * prec

    onorm = jnp.sqrt(jnp.sum(obs * obs, axis=1, keepdims=True))
    rinv = 1.0 / jnp.maximum(onorm, EPSILON)
    obsb_ref[...] = obs * (rinv * gp)

    raw = _dot_t(obs.astype(jnp.bfloat16), angs_ref[...])
    b = lax.bitcast_convert_type(raw, jnp.int32)
    packed = (b & andm_ref[...]) | orm_ref[...]
    pmax = jnp.max(lax.bitcast_convert_type(packed, jnp.float32), axis=1)
    pbest = lax.bitcast_convert_type(pmax, jnp.int32)
    bidx = (S - 1) - (pbest & jnp.int32(8191))
    bestv = lax.bitcast_convert_type(pbest & jnp.int32(-8192),
                                     jnp.float32) * rinv[:, 0]
    matched = (gp[:, 0] > RADIUS_THRESHOLD) & (bestv > MATCH_THRESHOLD)
    slots_ref[0, 0, :] = jnp.where(matched, bidx, -1).astype(jnp.int32)
    simsout_ref[0, 0, :] = jnp.where(matched, bestv, 0.0)


@functools.partial(jax.jit, static_argnames=())
def kernel(hidden, beliefs, active_mask, W_obs, w1, b1, w2, b2, w3, b3, w4, b4):
    B, T, H = hidden.shape
    D = W_obs.shape[0]
    Hq = w1.shape[0]
    S = beliefs.shape[0]
    N = B * T
    nblk = N // NB

    hid2d = hidden.reshape(N, H)
    wcat = jnp.concatenate([W_obs, w1, w3], axis=0).astype(jnp.bfloat16)
    wbd = jnp.zeros((2, 2 * Hq), jnp.float32)
    wbd = wbd.at[0, :Hq].set(w2[0]).at[1, Hq:].set(w4[0]).astype(jnp.bfloat16)
    b13 = jnp.concatenate([b1, b3]).reshape(1, 2 * Hq).astype(jnp.float32)
    b24 = jnp.concatenate([b2, b4]).reshape(1, 2).astype(jnp.float32)
    revcol = (S - 1 - jnp.arange(S, dtype=jnp.int32)).reshape(1, S)
    andm = jnp.where(active_mask, jnp.int32(-8192), jnp.int32(0)).reshape(1, S)
    orm = revcol | jnp.where(active_mask, jnp.int32(0),
                             jnp.int32(-2147483648)).reshape(1, S)

    grid = (nblk,)
    obsb, slots3, sims3 = pl.pallas_call(
        _fused_kernel,
        grid=grid,
        in_specs=[
            pl.BlockSpec((NB, H), lambda i: (i, 0)),
            pl.BlockSpec((D + 2 * Hq, H), lambda i: (0, 0)),
            pl.BlockSpec((2, 2 * Hq), lambda i: (0, 0)),
            pl.BlockSpec((1, 2 * Hq), lambda i: (0, 0)),
            pl.BlockSpec((1, 2), lambda i: (0, 0)),
            pl.BlockSpec((S, D), lambda i: (0, 0)),
            pl.BlockSpec((1, S), lambda i: (0, 0)),
            pl.BlockSpec((1, S), lambda i: (0, 0)),
        ],
        out_specs=[
            pl.BlockSpec((NB, D), lambda i: (i, 0)),
            pl.BlockSpec((1, 1, NB), lambda i: (i, 0, 0)),
            pl.BlockSpec((1, 1, NB), lambda i: (i, 0, 0)),
        ],
        out_shape=[
            jax.ShapeDtypeStruct((N, D), jnp.float32),
            jax.ShapeDtypeStruct((nblk, 1, NB), jnp.int32),
            jax.ShapeDtypeStruct((nblk, 1, NB), jnp.float32),
        ],
        scratch_shapes=[pltpu.VMEM((S, D), jnp.bfloat16)],
    )(hid2d, wcat, wbd, b13, b24, beliefs, andm, orm)

    return (obsb.reshape(B, T, D), slots3.reshape(N), sims3.reshape(N))
